# baseline (speedup 1.0000x reference)
import functools

import jax
import jax.numpy as jnp
from jax import lax
from jax.experimental import pallas as pl
from jax.experimental.pallas import tpu as pltpu
from jax.experimental.pallas import tpu_sc as plsc

VOCAB = 1000000
EMBED_DIM = 32
BATCH = 16384

NC = 2
NS = 16
NW = NC * NS
B_PER_W = BATCH // NW
IDX_CHUNK = 128
N_CHUNKS = B_PER_W // IDX_CHUNK


def _sc_kernel(table_hbm, idx_hbm, out_hbm, idx_v, rows_v, sem):
    wid = lax.axis_index("s") * NC + lax.axis_index("c")
    base = wid * B_PER_W

    pltpu.sync_copy(idx_hbm.at[wid], idx_v)

    copies = []
    for i in range(N_CHUNKS):
        copies.append(
            pltpu.async_copy(
                table_hbm.at[idx_v.at[i]],
                rows_v.at[pl.ds(i * IDX_CHUNK, IDX_CHUNK)],
                sem,
            )
        )
    for c in copies:
        c.wait()

    perms = [jax.lax.iota(jnp.int32, 16) ^ sh for sh in (1, 2, 4, 8)]
    _dn = lax.GatherDimensionNumbers(
        offset_dims=(), collapsed_slice_dims=(0,), start_index_map=(0,))

    def _permute(x, p):
        return lax.gather(x, p[:, None], _dn, (1,),
                          mode=lax.GatherScatterMode.PROMISE_IN_BOUNDS)

    def _allreduce(x, op):
        for p in perms:
            x = op(x, _permute(x, p))
        return x

    def body(r, carry):
        row0 = rows_v[r, pl.ds(0, 16)]
        row1 = rows_v[r, pl.ds(16, 16)]
        m = _allreduce(jnp.maximum(row0, row1), jnp.maximum)
        e0 = jnp.exp(row0 - m)
        e1 = jnp.exp(row1 - m)
        s = _allreduce(e0 + e1, jnp.add)
        inv = 1.0 / s
        rows_v[r, pl.ds(0, 16)] = e0 * inv
        rows_v[r, pl.ds(16, 16)] = e1 * inv
        return carry

    lax.fori_loop(0, B_PER_W, body, 0, unroll=4)

    pltpu.sync_copy(rows_v, out_hbm.at[pl.ds(base, B_PER_W)])


@jax.jit
def kernel(encoded, table):
    idx = encoded.astype(jnp.int32).reshape(NW, N_CHUNKS, IDX_CHUNK)
    run = functools.partial(
        pl.kernel,
        mesh=plsc.VectorSubcoreMesh(core_axis_name="c", subcore_axis_name="s"),
        out_type=jax.ShapeDtypeStruct((BATCH, EMBED_DIM), jnp.float32),
        scratch_types=[
            pltpu.VMEM((N_CHUNKS, IDX_CHUNK), jnp.int32),
            pltpu.VMEM((B_PER_W, EMBED_DIM), jnp.float32),
            pltpu.SemaphoreType.DMA,
        ],
        compiler_params=pltpu.CompilerParams(use_tc_tiling_on_sc=False),
    )(_sc_kernel)
    return run(table, idx)
